# baseline (speedup 1.0000x reference)
import functools

import jax
import jax.numpy as jnp
from jax.experimental import pallas as pl
from jax.experimental.pallas import tpu as pltpu


def _rcag_kernel(x_ref, w1_ref, w2_ref, wd_ref, bd_ref, wu_ref, bu_ref,
                 wf_ref, pmask_ref, out_ref, s_ref, *, H, W, C, nb, Wp):
    HWp = H * Wp
    x = x_ref[0]
    K9 = 9 * C

    offs = [dy * Wp + dx for dy in (-1, 0, 1) for dx in (-1, 0, 1)]

    pad = (jax.lax.broadcasted_iota(jnp.int32, (8, HWp), 0) == 0
           ).astype(jnp.bfloat16)
    s_ref[K9:K9 + 8, :] = pad
    for t, off in enumerate(offs):
        if off > 0:
            s_ref[t * C:(t + 1) * C, pl.ds(HWp - off, off)] = (
                jnp.zeros((C, off), jnp.bfloat16))
        elif off < 0:
            s_ref[t * C:(t + 1) * C, 0:-off] = jnp.zeros((C, -off),
                                                         jnp.bfloat16)

    def conv3x3(a_bf, w_ref, blk):
        for t, off in enumerate(offs):
            if off > 0:
                s_ref[t * C:(t + 1) * C, 0:HWp - off] = a_bf[:, off:]
            elif off < 0:
                s_ref[t * C:(t + 1) * C, pl.ds(-off, HWp + off)] = (
                    a_bf[:, :HWp + off])
            else:
                s_ref[t * C:(t + 1) * C, :] = a_bf
        return jnp.dot(w_ref[blk], s_ref[:],
                       preferred_element_type=jnp.float32)

    pm_bf = pmask_ref[0].astype(jnp.bfloat16)
    a = x
    for blk in range(nb):
        r = jnp.maximum(conv3x3(a.astype(jnp.bfloat16) * pm_bf, w1_ref, blk),
                        0.0)
        r = conv3x3(r.astype(jnp.bfloat16) * pm_bf, w2_ref, blk)
        y = jnp.sum(r * pmask_ref[0], axis=1, keepdims=True) * (1.0 / (H * W))
        d = jnp.maximum(jnp.sum(wd_ref[blk] * y, axis=0, keepdims=True)
                        + bd_ref[blk], 0.0)
        s = jax.nn.sigmoid(jnp.sum(wu_ref[blk] * d, axis=1, keepdims=True)
                           + bu_ref[blk])
        a = r * s + a

    res = conv3x3(a.astype(jnp.bfloat16) * pm_bf, wf_ref, 0)
    out_ref[0] = (res + x).astype(out_ref.dtype)


def _stack_weights(w, b, C):
    nb = w.shape[0]
    base = jnp.transpose(w, (0, 2, 1, 3)).reshape(nb, C, 9 * C)
    extra = jnp.zeros((nb, C, 8), w.dtype)
    extra = extra.at[:, :, 0].set(b.reshape(nb, C))
    return jnp.concatenate([base, extra], axis=2).astype(jnp.bfloat16)


def kernel(x, w1, b1, w2, b2, wd, bd, wu, bu, wf, bf):
    N, C, H, W = x.shape
    Wp = W + 2
    HWp = H * Wp
    nb = w1.shape[0]
    Cr = wd.shape[-1]

    w1s = _stack_weights(w1, b1, C)
    w2s = _stack_weights(w2, b2, C)
    wfs = _stack_weights(wf, bf.reshape(1, C, 1), C)

    colp = jnp.arange(HWp, dtype=jnp.int32) % Wp
    pmask = (colp < W).astype(jnp.float32).reshape(1, 1, HWp)

    x_pad = jnp.pad(x, ((0, 0), (0, 0), (0, 0), (0, Wp - W))
                    ).reshape(N, C, HWp)

    kernel_fn = functools.partial(_rcag_kernel, H=H, W=W, C=C, nb=nb, Wp=Wp)

    def full(shape):
        return pl.BlockSpec(shape, lambda n, _s=shape: (0,) * len(_s))

    out = pl.pallas_call(
        kernel_fn,
        out_shape=jax.ShapeDtypeStruct((N, C, HWp), x.dtype),
        grid_spec=pltpu.PrefetchScalarGridSpec(
            num_scalar_prefetch=0,
            grid=(N,),
            in_specs=[
                pl.BlockSpec((1, C, HWp), lambda n: (n, 0, 0)),
                full((nb, C, 9 * C + 8)),
                full((nb, C, 9 * C + 8)),
                full((nb, C, Cr)), full((nb, 1, Cr)),
                full((nb, C, Cr)), full((nb, C, 1)),
                full((1, C, 9 * C + 8)),
                full((1, 1, HWp)),
            ],
            out_specs=pl.BlockSpec((1, C, HWp), lambda n: (n, 0, 0)),
            scratch_shapes=[pltpu.VMEM((9 * C + 8, HWp), jnp.bfloat16)],
        ),
        compiler_params=pltpu.CompilerParams(dimension_semantics=("parallel",)),
    )(x_pad, w1s, w2s, wd, bd, wu, bu, wfs, pmask)
    return out.reshape(N, C, H, Wp)[..., :W]


# speedup vs baseline: 1.4519x; 1.4519x over previous
import functools

import jax
import jax.numpy as jnp
from jax.experimental import pallas as pl
from jax.experimental.pallas import tpu as pltpu


def _rcag_kernel(x_ref, w1_ref, w2_ref, wd_ref, bd_ref, wu_ref, bu_ref,
                 wf_ref, mcol_ref, out_ref, s_ref, *, H, W, C, nb):
    HW = H * W
    HW2 = 2 * HW

    pad = (jax.lax.broadcasted_iota(jnp.int32, (8, HW2), 0) == 0
           ).astype(jnp.bfloat16)
    s_ref[3 * C:3 * C + 8, :] = pad
    zW = jnp.zeros((C, W), jnp.bfloat16)
    for base in (0, HW):
        s_ref[0:C, pl.ds(base, W)] = zW
        s_ref[2 * C:3 * C, pl.ds(base + HW - W, W)] = zW

    def conv_pair(a0, a1, w_ref, blk):
        for base, af in ((0, a0), (HW, a1)):
            ab = af.astype(jnp.bfloat16)
            s_ref[0:C, pl.ds(base + W, HW - W)] = ab[:, :HW - W]
            s_ref[C:2 * C, pl.ds(base, HW)] = ab
            s_ref[2 * C:3 * C, pl.ds(base, HW - W)] = ab[:, W:]
        B = jnp.dot(w_ref[blk], s_ref[:], preferred_element_type=jnp.float32)
        comb = (B[C:2 * C]
                + pltpu.roll(B[0:C], 1, 1) * mcol_ref[0]
                + pltpu.roll(B[2 * C:3 * C], HW2 - 1, 1) * mcol_ref[1])
        return comb[:, :HW], comb[:, HW:]

    def calayer(r, blk):
        y = jnp.sum(r, axis=1, keepdims=True) * (1.0 / HW)
        d = jnp.maximum(jnp.sum(wd_ref[blk] * y, axis=0, keepdims=True)
                        + bd_ref[blk], 0.0)
        return jax.nn.sigmoid(jnp.sum(wu_ref[blk] * d, axis=1, keepdims=True)
                              + bu_ref[blk])

    x0 = x_ref[0]
    x1 = x_ref[1]
    a0, a1 = x0, x1
    for blk in range(nb):
        r0, r1 = conv_pair(a0, a1, w1_ref, blk)
        r0 = jnp.maximum(r0, 0.0)
        r1 = jnp.maximum(r1, 0.0)
        r0, r1 = conv_pair(r0, r1, w2_ref, blk)
        a0 = r0 * calayer(r0, blk) + a0
        a1 = r1 * calayer(r1, blk) + a1

    res0, res1 = conv_pair(a0, a1, wf_ref, 0)
    out_ref[0] = (res0 + x0).astype(out_ref.dtype)
    out_ref[1] = (res1 + x1).astype(out_ref.dtype)


def _stack_weights(w, b, C):
    nb = w.shape[0]
    base = jnp.transpose(w.reshape(nb, 3, 3, C, C),
                         (0, 2, 3, 1, 4)).reshape(nb, 3 * C, 3 * C)
    extra = jnp.zeros((nb, 3 * C, 8), w.dtype)
    extra = extra.at[:, C:2 * C, 0].set(b.reshape(nb, C))
    return jnp.concatenate([base, extra], axis=2).astype(jnp.bfloat16)


def kernel(x, w1, b1, w2, b2, wd, bd, wu, bu, wf, bf):
    N, C, H, W = x.shape
    HW = H * W
    nb = w1.shape[0]
    Cr = wd.shape[-1]

    w1s = _stack_weights(w1, b1, C)
    w2s = _stack_weights(w2, b2, C)
    wfs = _stack_weights(wf, bf.reshape(1, C, 1), C)

    col = jnp.arange(2 * HW, dtype=jnp.int32) % W
    mcol = jnp.stack([(col != 0).astype(jnp.float32),
                      (col != W - 1).astype(jnp.float32)]).reshape(2, 1, 2 * HW)

    kernel_fn = functools.partial(_rcag_kernel, H=H, W=W, C=C, nb=nb)

    def full(shape):
        return pl.BlockSpec(shape, lambda n, _s=shape: (0,) * len(_s))

    out = pl.pallas_call(
        kernel_fn,
        out_shape=jax.ShapeDtypeStruct((N, C, HW), x.dtype),
        grid_spec=pltpu.PrefetchScalarGridSpec(
            num_scalar_prefetch=0,
            grid=(N // 2,),
            in_specs=[
                pl.BlockSpec((2, C, HW), lambda n: (n, 0, 0)),
                full((nb, 3 * C, 3 * C + 8)),
                full((nb, 3 * C, 3 * C + 8)),
                full((nb, C, Cr)), full((nb, 1, Cr)),
                full((nb, C, Cr)), full((nb, C, 1)),
                full((1, 3 * C, 3 * C + 8)),
                full((2, 1, 2 * HW)),
            ],
            out_specs=pl.BlockSpec((2, C, HW), lambda n: (n, 0, 0)),
            scratch_shapes=[pltpu.VMEM((3 * C + 8, 2 * HW), jnp.bfloat16)],
        ),
        compiler_params=pltpu.CompilerParams(dimension_semantics=("parallel",)),
    )(x.reshape(N, C, HW),
      w1s, w2s, wd, bd, wu, bu, wfs, mcol)
    return out.reshape(N, C, H, W)


# speedup vs baseline: 1.9466x; 1.3407x over previous
import functools

import jax
import jax.numpy as jnp
from jax.experimental import pallas as pl
from jax.experimental.pallas import tpu as pltpu


def _rcag_kernel(x_ref, w1_ref, w2_ref, wd_ref, bd_ref, wu_ref, bu_ref,
                 wf_ref, mcol_ref, out_ref, s_ref, *, H, W, C, nb):
    HW = H * W
    x = x_ref[0]

    s_ref[0:C, 0:W] = jnp.zeros((C, W), jnp.bfloat16)
    s_ref[2 * C:3 * C, pl.ds(HW - W, W)] = jnp.zeros((C, W), jnp.bfloat16)
    pad = (jax.lax.broadcasted_iota(jnp.int32, (8, HW), 0) == 0
           ).astype(jnp.bfloat16)
    s_ref[3 * C:3 * C + 8, :] = pad

    NCH = 4
    CH = HW // NCH

    def conv3x3(a_bf, w_ref, blk):
        s_ref[0:C, pl.ds(W, HW - W)] = a_bf[:, :HW - W]
        s_ref[C:2 * C, :] = a_bf
        s_ref[2 * C:3 * C, 0:HW - W] = a_bf[:, W:]
        parts = []
        for c in range(NCH):
            B = jnp.dot(w_ref[blk], s_ref[:, c * CH:(c + 1) * CH],
                        preferred_element_type=jnp.float32)
            parts.append(B[C:2 * C]
                         + pltpu.roll(B[0:C], 1, 1) * mcol_ref[0, :, :CH]
                         + pltpu.roll(B[2 * C:3 * C], CH - 1, 1)
                         * mcol_ref[1, :, :CH])
        return jnp.concatenate(parts, axis=1)

    a = x
    for blk in range(nb):
        r = jnp.maximum(conv3x3(a.astype(jnp.bfloat16), w1_ref, blk), 0.0)
        r = conv3x3(r.astype(jnp.bfloat16), w2_ref, blk)
        y = jnp.sum(r, axis=1, keepdims=True) * (1.0 / HW)
        d = jnp.maximum(jnp.sum(wd_ref[blk] * y, axis=0, keepdims=True)
                        + bd_ref[blk], 0.0)
        s = jax.nn.sigmoid(jnp.sum(wu_ref[blk] * d, axis=1, keepdims=True)
                           + bu_ref[blk])
        a = r * s + a

    res = conv3x3(a.astype(jnp.bfloat16), wf_ref, 0)
    out_ref[0] = (res + x).astype(out_ref.dtype)


def _stack_weights(w, b, C):
    nb = w.shape[0]
    base = jnp.transpose(w.reshape(nb, 3, 3, C, C),
                         (0, 2, 3, 1, 4)).reshape(nb, 3 * C, 3 * C)
    extra = jnp.zeros((nb, 3 * C, 8), w.dtype)
    extra = extra.at[:, C:2 * C, 0].set(b.reshape(nb, C))
    return jnp.concatenate([base, extra], axis=2).astype(jnp.bfloat16)


def kernel(x, w1, b1, w2, b2, wd, bd, wu, bu, wf, bf):
    N, C, H, W = x.shape
    HW = H * W
    nb = w1.shape[0]
    Cr = wd.shape[-1]

    w1s = _stack_weights(w1, b1, C)
    w2s = _stack_weights(w2, b2, C)
    wfs = _stack_weights(wf, bf.reshape(1, C, 1), C)

    col = jnp.arange(HW, dtype=jnp.int32) % W
    mcol = jnp.stack([(col != 0).astype(jnp.float32),
                      (col != W - 1).astype(jnp.float32)]).reshape(2, 1, HW)

    kernel_fn = functools.partial(_rcag_kernel, H=H, W=W, C=C, nb=nb)

    def full(shape):
        return pl.BlockSpec(shape, lambda n, _s=shape: (0,) * len(_s))

    out = pl.pallas_call(
        kernel_fn,
        out_shape=jax.ShapeDtypeStruct((N, C, HW), x.dtype),
        grid_spec=pltpu.PrefetchScalarGridSpec(
            num_scalar_prefetch=0,
            grid=(N,),
            in_specs=[
                pl.BlockSpec((1, C, HW), lambda n: (n, 0, 0)),
                full((nb, 3 * C, 3 * C + 8)),
                full((nb, 3 * C, 3 * C + 8)),
                full((nb, C, Cr)), full((nb, 1, Cr)),
                full((nb, C, Cr)), full((nb, C, 1)),
                full((1, 3 * C, 3 * C + 8)),
                full((2, 1, HW)),
            ],
            out_specs=pl.BlockSpec((1, C, HW), lambda n: (n, 0, 0)),
            scratch_shapes=[pltpu.VMEM((3 * C + 8, HW), jnp.bfloat16)],
        ),
        compiler_params=pltpu.CompilerParams(dimension_semantics=("parallel",)),
    )(x.reshape(N, C, HW),
      w1s, w2s, wd, bd, wu, bu, wfs, mcol)
    return out.reshape(N, C, H, W)


# speedup vs baseline: 1.9908x; 1.0227x over previous
import functools

import jax
import jax.numpy as jnp
from jax.experimental import pallas as pl
from jax.experimental.pallas import tpu as pltpu


def _rcag_kernel(x_ref, w1_ref, w2_ref, wd_ref, bd_ref, wu_ref, bu_ref,
                 wf_ref, mcol_ref, out_ref, s_ref, *, H, W, C, nb):
    HW = H * W
    x = x_ref[0]

    s_ref[0:C, 0:W] = jnp.zeros((C, W), jnp.bfloat16)
    s_ref[2 * C:3 * C, pl.ds(HW - W, W)] = jnp.zeros((C, W), jnp.bfloat16)
    pad = (jax.lax.broadcasted_iota(jnp.int32, (8, HW), 0) == 0
           ).astype(jnp.bfloat16)
    s_ref[3 * C:3 * C + 8, :] = pad

    NCH = 8
    CH = HW // NCH

    def conv3x3(a_bf, w_ref, blk):
        s_ref[0:C, pl.ds(W, HW - W)] = a_bf[:, :HW - W]
        s_ref[C:2 * C, :] = a_bf
        s_ref[2 * C:3 * C, 0:HW - W] = a_bf[:, W:]
        parts = []
        for c in range(NCH):
            B = jnp.dot(w_ref[blk], s_ref[:, c * CH:(c + 1) * CH],
                        preferred_element_type=jnp.float32)
            parts.append(B[C:2 * C]
                         + pltpu.roll(B[0:C], 1, 1) * mcol_ref[0, :, :CH]
                         + pltpu.roll(B[2 * C:3 * C], CH - 1, 1)
                         * mcol_ref[1, :, :CH])
        return jnp.concatenate(parts, axis=1)

    a = x
    for blk in range(nb):
        r = jnp.maximum(conv3x3(a.astype(jnp.bfloat16), w1_ref, blk), 0.0)
        r = conv3x3(r.astype(jnp.bfloat16), w2_ref, blk)
        y = jnp.sum(r, axis=1, keepdims=True) * (1.0 / HW)
        d = jnp.maximum(jnp.sum(wd_ref[blk] * y, axis=0, keepdims=True)
                        + bd_ref[blk], 0.0)
        s = jax.nn.sigmoid(jnp.sum(wu_ref[blk] * d, axis=1, keepdims=True)
                           + bu_ref[blk])
        a = r * s + a

    res = conv3x3(a.astype(jnp.bfloat16), wf_ref, 0)
    out_ref[0] = (res + x).astype(out_ref.dtype)


def _stack_weights(w, b, C):
    nb = w.shape[0]
    base = jnp.transpose(w.reshape(nb, 3, 3, C, C),
                         (0, 2, 3, 1, 4)).reshape(nb, 3 * C, 3 * C)
    extra = jnp.zeros((nb, 3 * C, 8), w.dtype)
    extra = extra.at[:, C:2 * C, 0].set(b.reshape(nb, C))
    return jnp.concatenate([base, extra], axis=2).astype(jnp.bfloat16)


def kernel(x, w1, b1, w2, b2, wd, bd, wu, bu, wf, bf):
    N, C, H, W = x.shape
    HW = H * W
    nb = w1.shape[0]
    Cr = wd.shape[-1]

    w1s = _stack_weights(w1, b1, C)
    w2s = _stack_weights(w2, b2, C)
    wfs = _stack_weights(wf, bf.reshape(1, C, 1), C)

    col = jnp.arange(HW, dtype=jnp.int32) % W
    mcol = jnp.stack([(col != 0).astype(jnp.float32),
                      (col != W - 1).astype(jnp.float32)]).reshape(2, 1, HW)

    kernel_fn = functools.partial(_rcag_kernel, H=H, W=W, C=C, nb=nb)

    def full(shape):
        return pl.BlockSpec(shape, lambda n, _s=shape: (0,) * len(_s))

    out = pl.pallas_call(
        kernel_fn,
        out_shape=jax.ShapeDtypeStruct((N, C, HW), x.dtype),
        grid_spec=pltpu.PrefetchScalarGridSpec(
            num_scalar_prefetch=0,
            grid=(N,),
            in_specs=[
                pl.BlockSpec((1, C, HW), lambda n: (n, 0, 0)),
                full((nb, 3 * C, 3 * C + 8)),
                full((nb, 3 * C, 3 * C + 8)),
                full((nb, C, Cr)), full((nb, 1, Cr)),
                full((nb, C, Cr)), full((nb, C, 1)),
                full((1, 3 * C, 3 * C + 8)),
                full((2, 1, HW)),
            ],
            out_specs=pl.BlockSpec((1, C, HW), lambda n: (n, 0, 0)),
            scratch_shapes=[pltpu.VMEM((3 * C + 8, HW), jnp.bfloat16)],
        ),
        compiler_params=pltpu.CompilerParams(dimension_semantics=("parallel",)),
    )(x.reshape(N, C, HW),
      w1s, w2s, wd, bd, wu, bu, wfs, mcol)
    return out.reshape(N, C, H, W)


# speedup vs baseline: 1.9930x; 1.0011x over previous
import functools

import jax
import jax.numpy as jnp
from jax.experimental import pallas as pl
from jax.experimental.pallas import tpu as pltpu


def _rcag_kernel(x_ref, w1_ref, w2_ref, wd_ref, bd_ref, wu_ref, bu_ref,
                 wf_ref, mcol_ref, out_ref, s_ref, *, H, W, C, nb):
    HW = H * W
    x = x_ref[0]

    s_ref[0:C, 0:W] = jnp.zeros((C, W), jnp.bfloat16)
    s_ref[2 * C:3 * C, pl.ds(HW - W, W)] = jnp.zeros((C, W), jnp.bfloat16)
    pad = (jax.lax.broadcasted_iota(jnp.int32, (8, HW), 0) == 0
           ).astype(jnp.bfloat16)
    s_ref[3 * C:3 * C + 8, :] = pad

    NCH = 16
    CH = HW // NCH

    def conv3x3(a_bf, w_ref, blk):
        s_ref[0:C, pl.ds(W, HW - W)] = a_bf[:, :HW - W]
        s_ref[C:2 * C, :] = a_bf
        s_ref[2 * C:3 * C, 0:HW - W] = a_bf[:, W:]
        parts = []
        for c in range(NCH):
            B = jnp.dot(w_ref[blk], s_ref[:, c * CH:(c + 1) * CH],
                        preferred_element_type=jnp.float32)
            parts.append(B[C:2 * C]
                         + pltpu.roll(B[0:C], 1, 1) * mcol_ref[0, :, :CH]
                         + pltpu.roll(B[2 * C:3 * C], CH - 1, 1)
                         * mcol_ref[1, :, :CH])
        return jnp.concatenate(parts, axis=1)

    a = x
    for blk in range(nb):
        r = jnp.maximum(conv3x3(a.astype(jnp.bfloat16), w1_ref, blk), 0.0)
        r = conv3x3(r.astype(jnp.bfloat16), w2_ref, blk)
        y = jnp.sum(r, axis=1, keepdims=True) * (1.0 / HW)
        d = jnp.maximum(jnp.sum(wd_ref[blk] * y, axis=0, keepdims=True)
                        + bd_ref[blk], 0.0)
        s = jax.nn.sigmoid(jnp.sum(wu_ref[blk] * d, axis=1, keepdims=True)
                           + bu_ref[blk])
        a = r * s + a

    res = conv3x3(a.astype(jnp.bfloat16), wf_ref, 0)
    out_ref[0] = (res + x).astype(out_ref.dtype)


def _stack_weights(w, b, C):
    nb = w.shape[0]
    base = jnp.transpose(w.reshape(nb, 3, 3, C, C),
                         (0, 2, 3, 1, 4)).reshape(nb, 3 * C, 3 * C)
    extra = jnp.zeros((nb, 3 * C, 8), w.dtype)
    extra = extra.at[:, C:2 * C, 0].set(b.reshape(nb, C))
    return jnp.concatenate([base, extra], axis=2).astype(jnp.bfloat16)


def kernel(x, w1, b1, w2, b2, wd, bd, wu, bu, wf, bf):
    N, C, H, W = x.shape
    HW = H * W
    nb = w1.shape[0]
    Cr = wd.shape[-1]

    w1s = _stack_weights(w1, b1, C)
    w2s = _stack_weights(w2, b2, C)
    wfs = _stack_weights(wf, bf.reshape(1, C, 1), C)

    col = jnp.arange(HW, dtype=jnp.int32) % W
    mcol = jnp.stack([(col != 0).astype(jnp.float32),
                      (col != W - 1).astype(jnp.float32)]).reshape(2, 1, HW)

    kernel_fn = functools.partial(_rcag_kernel, H=H, W=W, C=C, nb=nb)

    def full(shape):
        return pl.BlockSpec(shape, lambda n, _s=shape: (0,) * len(_s))

    out = pl.pallas_call(
        kernel_fn,
        out_shape=jax.ShapeDtypeStruct((N, C, HW), x.dtype),
        grid_spec=pltpu.PrefetchScalarGridSpec(
            num_scalar_prefetch=0,
            grid=(N,),
            in_specs=[
                pl.BlockSpec((1, C, HW), lambda n: (n, 0, 0)),
                full((nb, 3 * C, 3 * C + 8)),
                full((nb, 3 * C, 3 * C + 8)),
                full((nb, C, Cr)), full((nb, 1, Cr)),
                full((nb, C, Cr)), full((nb, C, 1)),
                full((1, 3 * C, 3 * C + 8)),
                full((2, 1, HW)),
            ],
            out_specs=pl.BlockSpec((1, C, HW), lambda n: (n, 0, 0)),
            scratch_shapes=[pltpu.VMEM((3 * C + 8, HW), jnp.bfloat16)],
        ),
        compiler_params=pltpu.CompilerParams(dimension_semantics=("parallel",)),
    )(x.reshape(N, C, HW),
      w1s, w2s, wd, bd, wu, bu, wfs, mcol)
    return out.reshape(N, C, H, W)
